# Initial kernel scaffold; baseline (speedup 1.0000x reference)
import jax
import jax.numpy as jnp
from jax.experimental import pallas as pl


def kernel(cat0, cat1, cat2, Interaction, E_cat0, E_cat1, E_cat2, E_inter, W, b):
    raise NotImplementedError("write your pallas kernel here")


# speedup vs baseline: 4.7120x; 4.7120x over previous
import functools

import jax
import jax.numpy as jnp
from jax import lax
from jax.experimental import pallas as pl
from jax.experimental.pallas import tpu as pltpu
from jax.experimental.pallas import tpu_sc as plsc

B = 1024
L = 200
N = B * L
D = 128
NT = 3
HD = 384

_INFO = plsc.get_sparse_core_info()
NC = _INFO.num_cores
NS = _INFO.num_subcores
NW = NC * NS
ROWS_PER_W = N // NW
CH = 128
NCH = ROWS_PER_W // CH


def _sc_gather_concat(t0, t1, t2, idx):
    mesh = plsc.VectorSubcoreMesh(core_axis_name="c", subcore_axis_name="s")

    @functools.partial(
        pl.kernel,
        mesh=mesh,
        out_type=jax.ShapeDtypeStruct((N, NT * D), jnp.float32),
        scratch_types=[
            pltpu.VMEM((NCH, CH), jnp.int32),
            pltpu.VMEM((CH, D), jnp.float32),
            pltpu.SemaphoreType.DMA,
        ],
    )
    def gather_kernel(t0_h, t1_h, t2_h, idx_h, out_h, idx_v, buf, sem):
        wid = lax.axis_index("s") * NC + lax.axis_index("c")
        base = wid * ROWS_PER_W
        for t, tbl in enumerate((t0_h, t1_h, t2_h)):
            pltpu.sync_copy(idx_h.at[t, wid], idx_v)

            def chunk_body(j, _, tbl=tbl, t=t):
                pltpu.async_copy(tbl.at[idx_v.at[j]], buf, sem).wait()
                pltpu.sync_copy(
                    buf,
                    out_h.at[pl.ds(base + j * CH, CH), pl.ds(t * D, D)],
                )
                return 0

            lax.fori_loop(0, NCH, chunk_body, 0)

    return gather_kernel(t0, t1, t2, idx)


BN = 1024
NB = N // BN


def _tc_project(g, inter_col, e_inter, w, bias):
    def body(g_ref, it_ref, ei_ref, w_ref, b_ref, out_ref):
        gb = g_ref[...].astype(jnp.bfloat16)
        wb = w_ref[0:NT * D, :].astype(jnp.bfloat16)
        acc = jnp.dot(gb, wb, preferred_element_type=jnp.float32)
        p = jnp.dot(ei_ref[...], w_ref[NT * D:, :],
                    preferred_element_type=jnp.float32) + b_ref[...]
        iv = it_ref[...]
        acc = acc + jnp.where(iv == 0, 1.0, 0.0) * p[0:1, :]
        acc = acc + jnp.where(iv == 1, 1.0, 0.0) * p[1:2, :]
        acc = acc + jnp.where(iv == 2, 1.0, 0.0) * p[2:3, :]
        out_ref[...] = acc

    return pl.pallas_call(
        body,
        grid=(NB,),
        in_specs=[
            pl.BlockSpec((BN, NT * D), lambda i: (i, 0)),
            pl.BlockSpec((BN, 1), lambda i: (i, 0)),
            pl.BlockSpec((3, D), lambda i: (0, 0)),
            pl.BlockSpec((4 * D, HD), lambda i: (0, 0)),
            pl.BlockSpec((1, HD), lambda i: (0, 0)),
        ],
        out_specs=pl.BlockSpec((BN, HD), lambda i: (i, 0)),
        out_shape=jax.ShapeDtypeStruct((N, HD), jnp.float32),
        compiler_params=pltpu.CompilerParams(
            dimension_semantics=("arbitrary",),
        ),
    )(g, inter_col, e_inter, w, bias)


def kernel(cat0, cat1, cat2, Interaction, E_cat0, E_cat1, E_cat2, E_inter, W, b):
    idx = jnp.stack(
        [cat0.reshape(-1), cat1.reshape(-1), cat2.reshape(-1)]
    ).astype(jnp.int32).reshape(NT, NW, NCH, CH)
    g = _sc_gather_concat(E_cat0, E_cat1, E_cat2, idx)
    inter_col = Interaction.reshape(N, 1).astype(jnp.int32)
    x = _tc_project(g, inter_col, E_inter, W, b.reshape(1, HD))
    return (x.reshape(B, L, HD), cat0.shape[0])
